# baseline (speedup 1.0000x reference)
import functools

import jax
import jax.numpy as jnp
from jax import lax
from jax.experimental import pallas as pl
from jax.experimental.pallas import tpu as pltpu
from jax.experimental.pallas import tpu_sc as plsc

B = 128
SLEN = 8192
K_SEL = 128

NC = 2
NS = 16
L = 16
NW = NC * NS
ROWS_PER_W = B // NW
NVEC = SLEN // L
NSEG = 4
CHUNK = SLEN // NSEG
CVEC = CHUNK // L
SEG = CHUNK + 40 * L

_SIGN = jnp.int32(-2147483648)


def _map_keys(v):
    bi = lax.bitcast_convert_type(v, jnp.int32)
    s = lax.shift_right_arithmetic(bi, jnp.int32(31))
    u = lax.bitwise_xor(bi, lax.bitwise_or(s, _SIGN))
    return lax.bitcast_convert_type(u, jnp.uint32)


@functools.partial(
    pl.kernel,
    out_type=jax.ShapeDtypeStruct((B * SLEN,), jnp.float32),
    mesh=plsc.VectorSubcoreMesh(core_axis_name="c", subcore_axis_name="s"),
    compiler_params=pltpu.CompilerParams(needs_layout_passes=False),
    scratch_types=[
        pltpu.VMEM((ROWS_PER_W * SLEN,), jnp.float32),
        pltpu.VMEM((NSEG * SEG,), jnp.uint32),
        pltpu.VMEM((NSEG * SEG,), jnp.uint32),
        pltpu.VMEM((SLEN,), jnp.uint32),
    ],
)
def _topk_mask_sc(x_hbm, out_hbm, xf, ca, cb, uraw):
    wid = lax.axis_index("s") * NC + lax.axis_index("c")
    base = wid * ROWS_PER_W

    pltpu.sync_copy(x_hbm.at[pl.ds(base * SLEN, ROWS_PER_W * SLEN)], xf)

    one = jnp.ones((L,), jnp.int32)
    zero = jnp.zeros((L,), jnp.int32)
    zero_u = jnp.zeros((L,), jnp.uint32)
    tvec = jnp.full((L,), True)
    fvec = jnp.full((L,), False)

    def select_step(bit, state, src, dst):
        lo, cnt_hi, ns = state
        b0 = bit - 1
        s0 = jnp.full((L,), 1, jnp.uint32) << jnp.full(
            (L,), b0, dtype=jnp.uint32)
        m1 = lo + s0 + s0
        m2 = m1 + s0
        m3 = lo + s0
        nvs = [(n + (L - 1)) // L for n in ns]
        nv = jnp.maximum(jnp.maximum(nvs[0], nvs[1]),
                         jnp.maximum(nvs[2], nvs[3]))

        def cnt_body(j, accs):
            a1, a2, a3 = accs
            for i in range(NSEG):
                u = src[pl.ds(i * SEG + j * L, L)]
                valid = jnp.where(j < nvs[i], tvec, fvec)
                a1 = a1 + jnp.where((u >= m1) & valid, one, zero)
                a2 = a2 + jnp.where((u >= m2) & valid, one, zero)
                a3 = a3 + jnp.where((u >= m3) & valid, one, zero)
            return a1, a2, a3

        a1, a2, a3 = lax.fori_loop(0, nv, cnt_body, (zero,) * 3)
        c1 = cnt_hi + jnp.sum(a1)
        c2 = cnt_hi + jnp.sum(a2)
        c3 = cnt_hi + jnp.sum(a3)

        lo = jnp.where(
            c1 >= K_SEL,
            jnp.where(c2 >= K_SEL, m2, m1),
            jnp.where(c3 >= K_SEL, m3, lo),
        )
        cnt_hi = jnp.where(
            c1 >= K_SEL,
            jnp.where(c2 >= K_SEL, cnt_hi, c2),
            jnp.where(c3 >= K_SEL, c1, c3),
        )
        sh = jnp.full((L,), b0, dtype=jnp.uint32)
        want = lax.shift_right_logical(lo, sh)

        def cmp_body(j, poss):
            out = []
            for i in range(NSEG):
                u = src[pl.ds(i * SEG + j * L, L)]
                m = lax.shift_right_logical(u, sh) == want
                sel = m & jnp.where(j < nvs[i], tvec, fvec)
                plsc.store_compressed(dst.at[pl.ds(i * SEG + poss[i], L)],
                                      u, mask=sel)
                out.append(poss[i] + jnp.sum(jnp.where(sel, one, zero)))
            return tuple(out)

        poss = lax.fori_loop(0, nv, cmp_body, (jnp.int32(0),) * NSEG)
        for i in range(NSEG):
            dst[pl.ds(i * SEG + poss[i], L)] = zero_u

        return lo, cnt_hi, poss

    def row_body(r, _):
        rb = r * SLEN

        b_sign = jnp.full((L,), 0x80000000, dtype=jnp.uint32)
        b_hi = jnp.full((L,), 0xC0000000, dtype=jnp.uint32)
        b_lo = jnp.full((L,), 0x40000000, dtype=jnp.uint32)

        def cnt0_body(j, accs):
            a1, a2, a3 = accs
            for i in range(NSEG):
                v = xf[pl.ds(rb + i * CHUNK + j * L, L)]
                u = _map_keys(v)
                uraw[pl.ds(i * CHUNK + j * L, L)] = u
                a1 = a1 + jnp.where(u >= b_sign, one, zero)
                a2 = a2 + jnp.where(u >= b_hi, one, zero)
                a3 = a3 + jnp.where(u >= b_lo, one, zero)
            return a1, a2, a3

        a1, a2, a3 = plsc.parallel_loop(
            0, CVEC, unroll=4, carry=(zero,) * 3)(cnt0_body)
        c1 = jnp.sum(a1)
        c2a = jnp.sum(a2)
        c2b = jnp.sum(a3)

        lo = jnp.where(
            c1 >= K_SEL,
            jnp.where(c2a >= K_SEL, b_hi, b_sign),
            jnp.where(c2b >= K_SEL, b_lo, jnp.zeros((L,), jnp.uint32)),
        )
        cnt_hi = jnp.where(
            c1 >= K_SEL,
            jnp.where(c2a >= K_SEL, jnp.int32(0), c2a),
            jnp.where(c2b >= K_SEL, c1, c2b),
        )
        prefix2 = lax.shift_right_logical(lo, jnp.uint32(30))

        def cmp0_body(j, poss):
            out = []
            for i in range(NSEG):
                u = uraw[pl.ds(i * CHUNK + j * L, L)]
                sel = lax.shift_right_logical(u, jnp.uint32(30)) == prefix2
                plsc.store_compressed(ca.at[pl.ds(i * SEG + poss[i], L)],
                                      u, mask=sel)
                out.append(poss[i] + jnp.sum(jnp.where(sel, one, zero)))
            return tuple(out)

        poss = plsc.parallel_loop(
            0, CVEC, unroll=4, carry=(jnp.int32(0),) * NSEG)(cmp0_body)
        for i in range(NSEG):
            ca[pl.ds(i * SEG + poss[i], L)] = zero_u

        def tot(ns):
            return ns[0] + ns[1] + ns[2] + ns[3]

        def w_cond(carry):
            bit, (lo, cnt_hi, ns) = carry
            return (bit >= 0) & (tot(ns) > L)

        def w_body(carry):
            bit, state = carry
            state = select_step(bit, state, ca, cb)
            state = select_step(jnp.maximum(bit - 2, 1), state, cb, ca)
            return bit - 4, state

        init = (jnp.int32(29), (lo, cnt_hi, poss))
        _, (lo, cnt_hi, ns) = lax.while_loop(w_cond, w_body, init)

        def merge_body(i, pos):
            v = ca[pl.ds(i * SEG, L)]
            m = v != jnp.zeros((L,), jnp.uint32)
            plsc.store_compressed(cb.at[pl.ds(pos, L)], v, mask=m)
            return pos + jnp.sum(jnp.where(m, one, zero))

        posm = lax.fori_loop(0, NSEG, merge_body, jnp.int32(0))
        cb[pl.ds(posm, L)] = zero_u
        merged = cb[pl.ds(0, L)]
        sorted_k, _ = plsc.sort_key_val(merged, merged, descending=True)
        lanes = lax.iota(jnp.int32, L)
        k_idx = jnp.int32(K_SEL) - cnt_hi - 1
        small_thresh = jnp.max(
            jnp.where(lanes == k_idx, sorted_k, jnp.uint32(0)))

        lo = jnp.where(tot(ns) > L, lo, small_thresh)

        lo_i = lax.bitcast_convert_type(lo, jnp.int32)
        was_pos = lo_i < 0
        bits = jnp.where(
            was_pos,
            lax.bitwise_xor(lo_i, _SIGN),
            lax.bitwise_not(lo_i),
        )
        tf = lax.bitcast_convert_type(bits, jnp.float32)

        def mask_body(j):
            for i in range(NSEG):
                v = xf[pl.ds(rb + i * CHUNK + j * L, L)]
                xf[pl.ds(rb + i * CHUNK + j * L, L)] = jnp.where(
                    v >= tf, jnp.float32(1.0), jnp.float32(0.0)
                )

        plsc.parallel_loop(0, CVEC, unroll=4)(mask_body)
        return 0

    lax.fori_loop(0, ROWS_PER_W, row_body, 0)

    pltpu.sync_copy(xf, out_hbm.at[pl.ds(base * SLEN, ROWS_PER_W * SLEN)])


def kernel(logits):
    x = logits.reshape(B * SLEN)
    y = _topk_mask_sc(x)
    return y.reshape(B, SLEN, 1)


# speedup vs baseline: 1.0498x; 1.0498x over previous
import functools

import jax
import jax.numpy as jnp
from jax import lax
from jax.experimental import pallas as pl
from jax.experimental.pallas import tpu as pltpu
from jax.experimental.pallas import tpu_sc as plsc

B = 128
SLEN = 8192
K_SEL = 128

NC = 2
NS = 16
L = 16
NW = NC * NS
ROWS_PER_W = B // NW
NVEC = SLEN // L
NSEG = 4
CHUNK = SLEN // NSEG
CVEC = CHUNK // L
SEG = CHUNK + 40 * L

_SIGN = jnp.int32(-2147483648)


def _map_keys(v):
    bi = lax.bitcast_convert_type(v, jnp.int32)
    s = lax.shift_right_arithmetic(bi, jnp.int32(31))
    u = lax.bitwise_xor(bi, lax.bitwise_or(s, _SIGN))
    return lax.bitcast_convert_type(u, jnp.uint32)


@functools.partial(
    pl.kernel,
    out_type=jax.ShapeDtypeStruct((B * SLEN,), jnp.float32),
    mesh=plsc.VectorSubcoreMesh(core_axis_name="c", subcore_axis_name="s"),
    compiler_params=pltpu.CompilerParams(needs_layout_passes=False),
    scratch_types=[
        pltpu.VMEM((ROWS_PER_W * SLEN,), jnp.float32),
        pltpu.VMEM((NSEG * SEG,), jnp.uint32),
        pltpu.VMEM((NSEG * SEG,), jnp.uint32),
        pltpu.VMEM((SLEN,), jnp.uint32),
    ],
)
def _topk_mask_sc(x_hbm, out_hbm, xf, ca, cb, uraw):
    wid = lax.axis_index("s") * NC + lax.axis_index("c")
    base = wid * ROWS_PER_W

    pltpu.sync_copy(x_hbm.at[pl.ds(base * SLEN, ROWS_PER_W * SLEN)], xf)

    one = jnp.ones((L,), jnp.int32)
    zero = jnp.zeros((L,), jnp.int32)
    zero_u = jnp.zeros((L,), jnp.uint32)
    tvec = jnp.full((L,), True)
    fvec = jnp.full((L,), False)

    def select_step(bit, state, src, dst):
        lo, cnt_hi, ns = state
        b0 = bit - 1
        s0 = jnp.full((L,), 1, jnp.uint32) << jnp.full(
            (L,), b0, dtype=jnp.uint32)
        m1 = lo + s0 + s0
        m2 = m1 + s0
        m3 = lo + s0
        nvs = [(n + (L - 1)) // L for n in ns]
        nv = jnp.maximum(jnp.maximum(nvs[0], nvs[1]),
                         jnp.maximum(nvs[2], nvs[3]))

        def cnt_body(j, accs):
            a1, a2, a3 = accs
            for i in range(NSEG):
                u = src[pl.ds(i * SEG + j * L, L)]
                valid = jnp.where(j < nvs[i], tvec, fvec)
                a1 = a1 + jnp.where((u >= m1) & valid, one, zero)
                a2 = a2 + jnp.where((u >= m2) & valid, one, zero)
                a3 = a3 + jnp.where((u >= m3) & valid, one, zero)
            return a1, a2, a3

        a1, a2, a3 = lax.fori_loop(0, nv, cnt_body, (zero,) * 3)
        c1 = cnt_hi + jnp.sum(a1)
        c2 = cnt_hi + jnp.sum(a2)
        c3 = cnt_hi + jnp.sum(a3)

        lo = jnp.where(
            c1 >= K_SEL,
            jnp.where(c2 >= K_SEL, m2, m1),
            jnp.where(c3 >= K_SEL, m3, lo),
        )
        cnt_hi = jnp.where(
            c1 >= K_SEL,
            jnp.where(c2 >= K_SEL, cnt_hi, c2),
            jnp.where(c3 >= K_SEL, c1, c3),
        )
        sh = jnp.full((L,), b0, dtype=jnp.uint32)
        want = lax.shift_right_logical(lo, sh)

        def cmp_body(j, poss):
            out = []
            for i in range(NSEG):
                u = src[pl.ds(i * SEG + j * L, L)]
                m = lax.shift_right_logical(u, sh) == want
                sel = m & jnp.where(j < nvs[i], tvec, fvec)
                plsc.store_compressed(dst.at[pl.ds(i * SEG + poss[i], L)],
                                      u, mask=sel)
                out.append(poss[i] + jnp.sum(jnp.where(sel, one, zero)))
            return tuple(out)

        poss = lax.fori_loop(0, nv, cmp_body, (jnp.int32(0),) * NSEG)
        for i in range(NSEG):
            dst[pl.ds(i * SEG + poss[i], L)] = zero_u

        return lo, cnt_hi, poss

    def row_body(r, _):
        rb = r * SLEN

        b_sign = jnp.full((L,), 0x80000000, dtype=jnp.uint32)
        b_hi = jnp.full((L,), 0xC0000000, dtype=jnp.uint32)
        b_lo = jnp.full((L,), 0x40000000, dtype=jnp.uint32)

        def cnt0_body(j, accs):
            a1, a2, a3 = accs
            for i in range(NSEG):
                v = xf[pl.ds(rb + i * CHUNK + j * L, L)]
                u = _map_keys(v)
                uraw[pl.ds(i * CHUNK + j * L, L)] = u
                a1 = a1 + jnp.where(u >= b_sign, one, zero)
                a2 = a2 + jnp.where(u >= b_hi, one, zero)
                a3 = a3 + jnp.where(u >= b_lo, one, zero)
            return a1, a2, a3

        a1, a2, a3 = plsc.parallel_loop(
            0, CVEC, unroll=2, carry=(zero,) * 3)(cnt0_body)
        c1 = jnp.sum(a1)
        c2a = jnp.sum(a2)
        c2b = jnp.sum(a3)

        lo = jnp.where(
            c1 >= K_SEL,
            jnp.where(c2a >= K_SEL, b_hi, b_sign),
            jnp.where(c2b >= K_SEL, b_lo, jnp.zeros((L,), jnp.uint32)),
        )
        cnt_hi = jnp.where(
            c1 >= K_SEL,
            jnp.where(c2a >= K_SEL, jnp.int32(0), c2a),
            jnp.where(c2b >= K_SEL, c1, c2b),
        )
        prefix2 = lax.shift_right_logical(lo, jnp.uint32(30))

        def cmp0_body(j, poss):
            out = []
            for i in range(NSEG):
                u = uraw[pl.ds(i * CHUNK + j * L, L)]
                sel = lax.shift_right_logical(u, jnp.uint32(30)) == prefix2
                plsc.store_compressed(ca.at[pl.ds(i * SEG + poss[i], L)],
                                      u, mask=sel)
                out.append(poss[i] + jnp.sum(jnp.where(sel, one, zero)))
            return tuple(out)

        poss = plsc.parallel_loop(
            0, CVEC, unroll=2, carry=(jnp.int32(0),) * NSEG)(cmp0_body)
        for i in range(NSEG):
            ca[pl.ds(i * SEG + poss[i], L)] = zero_u

        def tot(ns):
            return ns[0] + ns[1] + ns[2] + ns[3]

        def w_cond(carry):
            bit, (lo, cnt_hi, ns) = carry
            return (bit >= 0) & (tot(ns) > L)

        def w_body(carry):
            bit, state = carry
            state = select_step(bit, state, ca, cb)
            state = select_step(jnp.maximum(bit - 2, 1), state, cb, ca)
            return bit - 4, state

        init = (jnp.int32(29), (lo, cnt_hi, poss))
        _, (lo, cnt_hi, ns) = lax.while_loop(w_cond, w_body, init)

        def merge_body(i, pos):
            v = ca[pl.ds(i * SEG, L)]
            m = v != jnp.zeros((L,), jnp.uint32)
            plsc.store_compressed(cb.at[pl.ds(pos, L)], v, mask=m)
            return pos + jnp.sum(jnp.where(m, one, zero))

        posm = lax.fori_loop(0, NSEG, merge_body, jnp.int32(0))
        cb[pl.ds(posm, L)] = zero_u
        merged = cb[pl.ds(0, L)]
        sorted_k, _ = plsc.sort_key_val(merged, merged, descending=True)
        lanes = lax.iota(jnp.int32, L)
        k_idx = jnp.int32(K_SEL) - cnt_hi - 1
        small_thresh = jnp.max(
            jnp.where(lanes == k_idx, sorted_k, jnp.uint32(0)))

        lo = jnp.where(tot(ns) > L, lo, small_thresh)

        lo_i = lax.bitcast_convert_type(lo, jnp.int32)
        was_pos = lo_i < 0
        bits = jnp.where(
            was_pos,
            lax.bitwise_xor(lo_i, _SIGN),
            lax.bitwise_not(lo_i),
        )
        tf = lax.bitcast_convert_type(bits, jnp.float32)

        def mask_body(j):
            for i in range(NSEG):
                v = xf[pl.ds(rb + i * CHUNK + j * L, L)]
                xf[pl.ds(rb + i * CHUNK + j * L, L)] = jnp.where(
                    v >= tf, jnp.float32(1.0), jnp.float32(0.0)
                )

        plsc.parallel_loop(0, CVEC, unroll=4)(mask_body)
        return 0

    lax.fori_loop(0, ROWS_PER_W, row_body, 0)

    pltpu.sync_copy(xf, out_hbm.at[pl.ds(base * SLEN, ROWS_PER_W * SLEN)])


def kernel(logits):
    x = logits.reshape(B * SLEN)
    y = _topk_mask_sc(x)
    return y.reshape(B, SLEN, 1)
